# baseline (speedup 1.0000x reference)
import jax
import jax.numpy as jnp
from jax import lax
from jax.experimental import pallas as pl
from jax.experimental.pallas import tpu as pltpu

_K = 100
_BIG = 1 << 30
_CB = 8


def _stage1_kernel(x_ref, s_ref, i_ref):
    x = x_ref[0]
    minf = jnp.float32(-jnp.inf)

    neg_row = jnp.full((_CB, 1, 128), minf, jnp.float32)
    neg_col = jnp.full((_CB, 128, 1), minf, jnp.float32)
    h = jnp.maximum(
        x,
        jnp.maximum(
            jnp.concatenate([x[:, :, 1:], neg_col], axis=2),
            jnp.concatenate([neg_col, x[:, :, :-1]], axis=2),
        ),
    )
    hmax = jnp.maximum(
        h,
        jnp.maximum(
            jnp.concatenate([h[:, 1:, :], neg_row], axis=1),
            jnp.concatenate([neg_row, h[:, :-1, :]], axis=1),
        ),
    )
    hm = jnp.where(hmax == x, x, jnp.float32(0.0))

    r4 = hm.reshape(_CB, 64, 2, 128)
    a = r4[:, :, 0, :]
    b = r4[:, :, 1, :]
    la = (
        lax.broadcasted_iota(jnp.int32, (_CB, 64, 128), 1) * 256
        + lax.broadcasted_iota(jnp.int32, (_CB, 64, 128), 2)
    )
    keep_a = a >= b
    cand = jnp.where(keep_a, a, b)
    lin = jnp.where(keep_a, la, la + 128)
    lane = lax.broadcasted_iota(jnp.int32, (1, 128), 1)

    def body(i, carry):
        xm, sacc, iacc = carry
        m = jnp.max(xm, axis=(1, 2), keepdims=True)
        p = jnp.min(jnp.where(xm == m, lin, _BIG), axis=(1, 2), keepdims=True)
        xm = jnp.where(lin == p, jnp.float32(-1.0), xm)
        sacc = jnp.where(lane == i, m.reshape(_CB, 1), sacc)
        iacc = jnp.where(lane == i, p.reshape(_CB, 1), iacc)
        return xm, sacc, iacc

    _, sacc, iacc = lax.fori_loop(
        0, _K, body,
        (cand, jnp.zeros((_CB, 128), jnp.float32), jnp.zeros((_CB, 128), jnp.int32)),
    )
    s_ref[0] = sacc
    i_ref[0] = iacc


def _stage2_kernel(s_ref, i_ref, oi_ref, oc_ref, oy_ref, ox_ref):
    s = s_ref[0]
    idx = i_ref[0]
    col = lax.broadcasted_iota(jnp.int32, (80, 128), 1)
    row = lax.broadcasted_iota(jnp.int32, (80, 128), 0)
    valid = col < _K
    s = jnp.where(valid, s, jnp.float32(-2.0))
    flat = jnp.where(valid, row * _K + col, _BIG)
    lane = lax.broadcasted_iota(jnp.int32, (1, 128), 1)

    def body(i, carry):
        sm, iacc, cacc = carry
        m = jnp.max(sm)
        p = jnp.min(jnp.where(sm == m, flat, _BIG))
        hit = flat == p
        iv = jnp.sum(jnp.where(hit, idx, 0))
        sm = jnp.where(hit, jnp.float32(-2.0), sm)
        iacc = jnp.where(lane == i, iv, iacc)
        cacc = jnp.where(lane == i, p // _K, cacc)
        return sm, iacc, cacc

    _, iacc, cacc = lax.fori_loop(
        0, _K, body,
        (s, jnp.zeros((1, 128), jnp.int32), jnp.zeros((1, 128), jnp.int32)),
    )
    oi_ref[0] = iacc
    oc_ref[0] = cacc
    oy_ref[0] = (iacc // 128).astype(jnp.float32)
    ox_ref[0] = (iacc % 128).astype(jnp.float32)


def kernel(heatmap):
    BS, C, H, W = heatmap.shape

    s1_scores, s1_idx = pl.pallas_call(
        _stage1_kernel,
        grid=(BS, C // _CB),
        in_specs=[pl.BlockSpec((1, _CB, H, W), lambda b, c: (b, c, 0, 0))],
        out_specs=[
            pl.BlockSpec((1, _CB, 128), lambda b, c: (b, c, 0)),
            pl.BlockSpec((1, _CB, 128), lambda b, c: (b, c, 0)),
        ],
        out_shape=[
            jax.ShapeDtypeStruct((BS, C, 128), jnp.float32),
            jax.ShapeDtypeStruct((BS, C, 128), jnp.int32),
        ],
        compiler_params=pltpu.CompilerParams(
            dimension_semantics=("parallel", "parallel")
        ),
    )(heatmap)

    top_idx, top_cls, top_ys, top_xs = pl.pallas_call(
        _stage2_kernel,
        grid=(BS,),
        in_specs=[
            pl.BlockSpec((1, C, 128), lambda b: (b, 0, 0)),
            pl.BlockSpec((1, C, 128), lambda b: (b, 0, 0)),
        ],
        out_specs=[pl.BlockSpec((1, 1, 128), lambda b: (b, 0, 0))] * 4,
        out_shape=[
            jax.ShapeDtypeStruct((BS, 1, 128), jnp.int32),
            jax.ShapeDtypeStruct((BS, 1, 128), jnp.int32),
            jax.ShapeDtypeStruct((BS, 1, 128), jnp.float32),
            jax.ShapeDtypeStruct((BS, 1, 128), jnp.float32),
        ],
    )(s1_scores, s1_idx)

    return (
        s1_scores[:, :, :_K],
        top_idx[:, 0, :_K],
        top_cls[:, 0, :_K],
        top_ys[:, 0, :_K],
        top_xs[:, 0, :_K],
    )


# speedup vs baseline: 2.5194x; 2.5194x over previous
import jax
import jax.numpy as jnp
from jax import lax
from jax.experimental import pallas as pl
from jax.experimental.pallas import tpu as pltpu

_K = 100
_BIG = 1 << 30
_CB = 8


def _stage1_kernel(x_ref, s_ref, i_ref):
    x = x_ref[0]
    minf = jnp.float32(-jnp.inf)

    neg_row = jnp.full((_CB, 1, 128), minf, jnp.float32)
    neg_col = jnp.full((_CB, 128, 1), minf, jnp.float32)
    h = jnp.maximum(
        x,
        jnp.maximum(
            jnp.concatenate([x[:, :, 1:], neg_col], axis=2),
            jnp.concatenate([neg_col, x[:, :, :-1]], axis=2),
        ),
    )
    hmax = jnp.maximum(
        h,
        jnp.maximum(
            jnp.concatenate([h[:, 1:, :], neg_row], axis=1),
            jnp.concatenate([neg_row, h[:, :-1, :]], axis=1),
        ),
    )
    hm = jnp.where(hmax == x, x, jnp.float32(0.0))

    f32 = jnp.float32
    rsel = lax.broadcasted_iota(jnp.int32, (64, 128), 0)
    ksel = lax.broadcasted_iota(jnp.int32, (64, 128), 1)
    pe = jnp.broadcast_to(
        (ksel == 2 * rsel).astype(f32)[None], (_CB, 64, 128))
    po = jnp.broadcast_to(
        (ksel == 2 * rsel + 1).astype(f32)[None], (_CB, 64, 128))
    dn = (((2,), (1,)), ((0,), (0,)))
    a = lax.dot_general(pe, hm, dn, preferred_element_type=f32, precision=lax.Precision.HIGHEST)
    b = lax.dot_general(po, hm, dn, preferred_element_type=f32, precision=lax.Precision.HIGHEST)
    la = (
        lax.broadcasted_iota(jnp.int32, (_CB, 64, 128), 1) * 256
        + lax.broadcasted_iota(jnp.int32, (_CB, 64, 128), 2)
    )
    keep_a = a >= b
    v1 = jnp.where(keep_a, a, b)
    l1 = jnp.where(keep_a, la, la + 128).astype(f32)

    kq = lax.broadcasted_iota(jnp.int32, (128, 64), 0)
    cq = lax.broadcasted_iota(jnp.int32, (128, 64), 1)
    qe = jnp.broadcast_to((kq == 2 * cq).astype(f32)[None], (_CB, 128, 64))
    qo = jnp.broadcast_to((kq == 2 * cq + 1).astype(f32)[None], (_CB, 128, 64))
    a2 = lax.dot_general(v1, qe, dn, preferred_element_type=f32, precision=lax.Precision.HIGHEST)
    b2 = lax.dot_general(v1, qo, dn, preferred_element_type=f32, precision=lax.Precision.HIGHEST)
    la2 = lax.dot_general(l1, qe, dn, preferred_element_type=f32, precision=lax.Precision.HIGHEST)
    lb2 = lax.dot_general(l1, qo, dn, preferred_element_type=f32, precision=lax.Precision.HIGHEST)
    keep2 = a2 >= b2
    v2 = jnp.where(keep2, a2, b2)
    l2 = (jnp.where(keep2, la2, lb2) + f32(0.5)).astype(jnp.int32)

    pv = jnp.concatenate([v2[:4], v2[4:]], axis=2)
    pl2 = jnp.concatenate([l2[:4], l2[4:]], axis=2)

    lane3 = lax.broadcasted_iota(jnp.int32, (4, 1, 128), 2)
    left = lane3 < 64
    lane = lax.broadcasted_iota(jnp.int32, (1, 128), 1)

    def body(i, carry):
        xm, sacc, iacc = carry
        mr = jnp.max(xm, axis=1, keepdims=True)
        ml = jnp.max(jnp.where(left, mr, f32(-3.0)), axis=2, keepdims=True)
        mrr = jnp.max(jnp.where(left, f32(-3.0), mr), axis=2, keepdims=True)
        m = jnp.where(left, ml, mrr)
        cnd = jnp.where(xm == m, pl2, _BIG)
        pr = jnp.min(cnd, axis=1, keepdims=True)
        plft = jnp.min(jnp.where(left, pr, _BIG), axis=2, keepdims=True)
        prgt = jnp.min(jnp.where(left, _BIG, pr), axis=2, keepdims=True)
        p = jnp.where(left, plft, prgt)
        xm = jnp.where(pl2 == p, f32(-1.0), xm)
        scol = jnp.concatenate(
            [ml.reshape(4, 1), mrr.reshape(4, 1)], axis=0)
        pcol = jnp.concatenate(
            [plft.reshape(4, 1), prgt.reshape(4, 1)], axis=0)
        sacc = jnp.where(lane == i, scol, sacc)
        iacc = jnp.where(lane == i, pcol, iacc)
        return xm, sacc, iacc

    _, sacc, iacc = lax.fori_loop(
        0, _K, body,
        (pv, jnp.zeros((_CB, 128), jnp.float32), jnp.zeros((_CB, 128), jnp.int32)),
    )
    s_ref[0] = sacc
    i_ref[0] = iacc


def _stage2_kernel(s_ref, i_ref, oi_ref, oc_ref, oy_ref, ox_ref):
    s = s_ref[0]
    idx = i_ref[0]
    col = lax.broadcasted_iota(jnp.int32, (80, 128), 1)
    row = lax.broadcasted_iota(jnp.int32, (80, 128), 0)
    valid = col < _K
    s = jnp.where(valid, s, jnp.float32(-2.0))
    flat = jnp.where(valid, row * _K + col, _BIG)
    lane = lax.broadcasted_iota(jnp.int32, (1, 128), 1)

    def body(i, carry):
        sm, iacc, cacc = carry
        m = jnp.max(sm)
        p = jnp.min(jnp.where(sm == m, flat, _BIG))
        hit = flat == p
        iv = jnp.sum(jnp.where(hit, idx, 0))
        sm = jnp.where(hit, jnp.float32(-2.0), sm)
        iacc = jnp.where(lane == i, iv, iacc)
        cacc = jnp.where(lane == i, p // _K, cacc)
        return sm, iacc, cacc

    _, iacc, cacc = lax.fori_loop(
        0, _K, body,
        (s, jnp.zeros((1, 128), jnp.int32), jnp.zeros((1, 128), jnp.int32)),
    )
    oi_ref[0] = iacc
    oc_ref[0] = cacc
    oy_ref[0] = (iacc // 128).astype(jnp.float32)
    ox_ref[0] = (iacc % 128).astype(jnp.float32)


def kernel(heatmap):
    BS, C, H, W = heatmap.shape

    s1_scores, s1_idx = pl.pallas_call(
        _stage1_kernel,
        grid=(BS, C // _CB),
        in_specs=[pl.BlockSpec((1, _CB, H, W), lambda b, c: (b, c, 0, 0))],
        out_specs=[
            pl.BlockSpec((1, _CB, 128), lambda b, c: (b, c, 0)),
            pl.BlockSpec((1, _CB, 128), lambda b, c: (b, c, 0)),
        ],
        out_shape=[
            jax.ShapeDtypeStruct((BS, C, 128), jnp.float32),
            jax.ShapeDtypeStruct((BS, C, 128), jnp.int32),
        ],
        compiler_params=pltpu.CompilerParams(
            dimension_semantics=("parallel", "parallel")
        ),
    )(heatmap)

    top_idx, top_cls, top_ys, top_xs = pl.pallas_call(
        _stage2_kernel,
        grid=(BS,),
        in_specs=[
            pl.BlockSpec((1, C, 128), lambda b: (b, 0, 0)),
            pl.BlockSpec((1, C, 128), lambda b: (b, 0, 0)),
        ],
        out_specs=[pl.BlockSpec((1, 1, 128), lambda b: (b, 0, 0))] * 4,
        out_shape=[
            jax.ShapeDtypeStruct((BS, 1, 128), jnp.int32),
            jax.ShapeDtypeStruct((BS, 1, 128), jnp.int32),
            jax.ShapeDtypeStruct((BS, 1, 128), jnp.float32),
            jax.ShapeDtypeStruct((BS, 1, 128), jnp.float32),
        ],
    )(s1_scores, s1_idx)

    return (
        s1_scores[:, :, :_K],
        top_idx[:, 0, :_K],
        top_cls[:, 0, :_K],
        top_ys[:, 0, :_K],
        top_xs[:, 0, :_K],
    )
